# baseline (speedup 1.0000x reference)
import functools

import jax
import jax.numpy as jnp
from jax import lax
from jax.experimental import pallas as pl
from jax.experimental.pallas import tpu as pltpu
from jax.experimental.pallas import tpu_sc as plsc

N = 10000
E = 320000
D = 128
K = 2

NS = 16
CHUNK = 128
EROWS = E // CHUNK
RSTD = 156
RLAST = EROWS - 15 * RSTD
RPT = 160
EPT_PAD = RPT * CHUNK
NPAD = 10240
SLICE = NPAD // NS
OSLICE = 640
OLAST = N - 15 * OSLICE
VL = 16
NCH = 8
CROWS = RPT // NCH
CSIZE = CROWS * CHUNK


def _matvec_body(x_ref, w_ref, o_ref):
    o_ref[...] = jax.lax.dot_general(
        w_ref[...], x_ref[...],
        dimension_numbers=(((1,), (1,)), ((), ())),
        preferred_element_type=jnp.float32).reshape(N)


def _matvec(x, w2):
    return pl.pallas_call(
        _matvec_body,
        in_specs=[
            pl.BlockSpec((N, D), lambda: (0, 0)),
            pl.BlockSpec((1, D), lambda: (0, 0)),
        ],
        out_specs=pl.BlockSpec((N,), lambda: (0,)),
        out_shape=jax.ShapeDtypeStruct((N,), jnp.float32),
    )(x, w2)


_mesh = plsc.VectorSubcoreMesh(
    core_axis_name="c", subcore_axis_name="s", num_cores=1)


@functools.partial(
    pl.kernel,
    out_type=jax.ShapeDtypeStruct((N,), jnp.float32),
    mesh=_mesh,
    compiler_params=pltpu.CompilerParams(
        use_tc_tiling_on_sc=False, needs_layout_passes=False),
    scratch_types=[
        pltpu.VMEM((NPAD,), jnp.float32),
        pltpu.VMEM((RPT, CHUNK), jnp.int32),
        pltpu.VMEM((RPT, CHUNK), jnp.int32),
        pltpu.VMEM((RPT, CHUNK), jnp.float32),
        pltpu.VMEM((EPT_PAD,), jnp.float32),
        pltpu.VMEM((EPT_PAD,), jnp.int32),
        pltpu.VMEM((SLICE,), jnp.float32),
        pltpu.VMEM_SHARED((NPAD,), jnp.float32),
        pltpu.VMEM_SHARED((NPAD,), jnp.float32),
        pltpu.SemaphoreType.DMA,
        pltpu.SemaphoreType.DMA,
    ],
)
def _sgc_sc(y0_hbm, ei_hbm, ew_hbm, out_hbm,
            y_l, src_l, dst_l, ew_l, msg_f, dst_f, sbuf,
            acc0, acc1, ssem, sem):
    sid = lax.axis_index("s")
    last = sid == NS - 1
    nreal = jnp.where(last, RLAST, RSTD)
    row0 = sid * RSTD

    @pl.when(jnp.logical_not(last))
    def _():
        pltpu.async_copy(ei_hbm.at[pl.ds(row0, RSTD), 0],
                         src_l.at[pl.ds(0, RSTD)], ssem)
        pltpu.async_copy(ei_hbm.at[pl.ds(row0, RSTD), 1],
                         dst_l.at[pl.ds(0, RSTD)], ssem)
        pltpu.async_copy(ew_hbm.at[pl.ds(row0, RSTD)],
                         ew_l.at[pl.ds(0, RSTD)], ssem)

    @pl.when(last)
    def _():
        pltpu.async_copy(ei_hbm.at[pl.ds(row0, RLAST), 0], src_l, ssem)
        pltpu.async_copy(ei_hbm.at[pl.ds(row0, RLAST), 1], dst_l, ssem)
        pltpu.async_copy(ew_hbm.at[pl.ds(row0, RLAST)], ew_l, ssem)
    pltpu.async_copy(y0_hbm, y_l.at[pl.ds(0, N)], ssem)

    def _zero(i, _):
        sbuf[pl.ds(i * VL, VL)] = jnp.zeros((VL,), jnp.float32)
        return 0

    lax.fori_loop(0, SLICE // VL, _zero, 0)

    @pl.when(jnp.logical_not(last))
    def _():
        for _i in range(2):
            pltpu.make_async_copy(ei_hbm.at[pl.ds(0, RSTD), 0],
                                  src_l.at[pl.ds(0, RSTD)], ssem).wait()
        pltpu.make_async_copy(ew_hbm.at[pl.ds(0, RSTD)],
                              ew_l.at[pl.ds(0, RSTD)], ssem).wait()

    @pl.when(last)
    def _():
        for _i in range(2):
            pltpu.make_async_copy(ei_hbm.at[pl.ds(0, RLAST), 0],
                                  src_l, ssem).wait()
        pltpu.make_async_copy(ew_hbm.at[pl.ds(0, RLAST)], ew_l, ssem).wait()
    pltpu.make_async_copy(y0_hbm, y_l.at[pl.ds(0, N)], ssem).wait()

    def _fill(j, _):
        for g in range(CHUNK // VL):
            src_l[j, pl.ds(g * VL, VL)] = jnp.zeros((VL,), jnp.int32)
            ew_l[j, pl.ds(g * VL, VL)] = jnp.zeros((VL,), jnp.float32)
            dst_l[j, pl.ds(g * VL, VL)] = jnp.full((VL,), N, jnp.int32)
        return 0

    lax.fori_loop(nreal, RPT, _fill, 0)

    pltpu.sync_copy(sbuf, acc0.at[pl.ds(sid * SLICE, SLICE)])
    pltpu.sync_copy(sbuf, acc1.at[pl.ds(sid * SLICE, SLICE)])
    plsc.subcore_barrier()

    for hop in range(K):
        acc = acc0 if hop == 0 else acc1

        def _row(j, _):
            for g in range(CHUNK // VL):
                s16 = src_l[j, pl.ds(g * VL, VL)]
                e16 = ew_l[j, pl.ds(g * VL, VL)]
                vals = plsc.load_gather(y_l, [s16])
                msg_f[pl.ds(j * CHUNK + g * VL, VL)] = vals * e16
                if hop == 0:
                    dst_f[pl.ds(j * CHUNK + g * VL, VL)] = \
                        dst_l[j, pl.ds(g * VL, VL)]
            return 0

        def _chunk(c, _):
            lax.fori_loop(c * CROWS, (c + 1) * CROWS, _row, 0)
            pltpu.async_copy(
                msg_f.at[pl.ds(c * CSIZE, CSIZE)],
                acc.at[dst_f.at[pl.ds(c * CSIZE, CSIZE)]],
                sem, add=True)
            return 0

        lax.fori_loop(0, NCH, _chunk, 0)

        def _drain(c, _):
            pltpu.make_async_copy(
                msg_f.at[pl.ds(0, CSIZE)],
                acc.at[dst_f.at[pl.ds(0, CSIZE)]], sem).wait()
            return 0

        lax.fori_loop(0, NCH, _drain, 0)
        plsc.subcore_barrier()
        if hop + 1 < K:
            @pl.when(jnp.logical_not(last))
            def _():
                pltpu.sync_copy(acc.at[pl.ds(sid * OSLICE, OSLICE)],
                                out_hbm.at[pl.ds(sid * OSLICE, OSLICE)])

            @pl.when(last)
            def _():
                pltpu.sync_copy(acc.at[pl.ds(15 * OSLICE, OLAST)],
                                out_hbm.at[pl.ds(15 * OSLICE, OLAST)])
            plsc.subcore_barrier()
            pltpu.sync_copy(out_hbm, y_l.at[pl.ds(0, N)])

    def _sig(i, _):
        z = sbuf[pl.ds(i * VL, VL)]
        sbuf[pl.ds(i * VL, VL)] = 1.0 / (1.0 + jnp.exp(-z))
        return 0

    @pl.when(jnp.logical_not(last))
    def _():
        pltpu.sync_copy(acc1.at[pl.ds(sid * OSLICE, OSLICE)],
                        sbuf.at[pl.ds(0, OSLICE)])
        lax.fori_loop(0, OSLICE // VL, _sig, 0)
        pltpu.sync_copy(sbuf.at[pl.ds(0, OSLICE)],
                        out_hbm.at[pl.ds(sid * OSLICE, OSLICE)])

    @pl.when(last)
    def _():
        pltpu.sync_copy(acc1.at[pl.ds(15 * OSLICE, OLAST)],
                        sbuf.at[pl.ds(0, OLAST)])
        lax.fori_loop(0, OLAST // VL, _sig, 0)
        pltpu.sync_copy(sbuf.at[pl.ds(0, OLAST)],
                        out_hbm.at[pl.ds(15 * OSLICE, OLAST)])


def kernel(x, edge_index, edge_weight, w):
    ei3 = edge_index.reshape(2, EROWS, CHUNK).transpose(1, 0, 2)
    ew2d = edge_weight.reshape(EROWS, CHUNK)
    y0 = _matvec(x, w.reshape(1, D))
    res = _sgc_sc(y0, ei3, ew2d)
    return res.reshape(N, 1)


# speedup vs baseline: 1.1775x; 1.1775x over previous
import functools

import jax
import jax.numpy as jnp
from jax import lax
from jax.experimental import pallas as pl
from jax.experimental.pallas import tpu as pltpu
from jax.experimental.pallas import tpu_sc as plsc

N = 10000
E = 320000
D = 128
K = 2

NS = 16
CHUNK = 128
EROWS = E // CHUNK
RSTD = 156
RLAST = EROWS - 15 * RSTD
RPT = 160
EPT_PAD = RPT * CHUNK
NPAD = 10240
SLICE = NPAD // NS
OSLICE = 640
OLAST = N - 15 * OSLICE
VL = 16
NCH = 8
CROWS = RPT // NCH
CSIZE = CROWS * CHUNK


def _matvec_body(x_ref, w_ref, o_ref):
    o_ref[...] = jax.lax.dot_general(
        w_ref[...], x_ref[...],
        dimension_numbers=(((1,), (1,)), ((), ())),
        preferred_element_type=jnp.float32).reshape(N)


def _matvec(x, w2):
    return pl.pallas_call(
        _matvec_body,
        in_specs=[
            pl.BlockSpec((N, D), lambda: (0, 0)),
            pl.BlockSpec((1, D), lambda: (0, 0)),
        ],
        out_specs=pl.BlockSpec((N,), lambda: (0,)),
        out_shape=jax.ShapeDtypeStruct((N,), jnp.float32),
    )(x, w2)


_mesh = plsc.VectorSubcoreMesh(
    core_axis_name="c", subcore_axis_name="s", num_cores=1)


@functools.partial(
    pl.kernel,
    out_type=jax.ShapeDtypeStruct((N,), jnp.float32),
    mesh=_mesh,
    compiler_params=pltpu.CompilerParams(
        use_tc_tiling_on_sc=False, needs_layout_passes=False),
    scratch_types=[
        pltpu.VMEM((NPAD,), jnp.float32),
        pltpu.VMEM((RPT, CHUNK), jnp.int32),
        pltpu.VMEM((RPT, CHUNK), jnp.int32),
        pltpu.VMEM((RPT, CHUNK), jnp.float32),
        pltpu.VMEM((EPT_PAD,), jnp.float32),
        pltpu.VMEM((EPT_PAD,), jnp.int32),
        pltpu.VMEM((SLICE,), jnp.float32),
        pltpu.VMEM_SHARED((NPAD,), jnp.float32),
        pltpu.VMEM_SHARED((NPAD,), jnp.float32),
        pltpu.SemaphoreType.DMA,
        pltpu.SemaphoreType.DMA,
    ],
)
def _sgc_sc(y0_hbm, ei_hbm, ew_hbm, out_hbm,
            y_l, src_l, dst_l, ew_l, msg_f, dst_f, sbuf,
            acc0, acc1, ssem, sem):
    sid = lax.axis_index("s")
    last = sid == NS - 1
    nreal = jnp.where(last, RLAST, RSTD)
    row0 = sid * RSTD

    @pl.when(jnp.logical_not(last))
    def _():
        pltpu.async_copy(ei_hbm.at[pl.ds(row0, RSTD), 0],
                         src_l.at[pl.ds(0, RSTD)], ssem)
        pltpu.async_copy(ei_hbm.at[pl.ds(row0, RSTD), 1],
                         dst_l.at[pl.ds(0, RSTD)], ssem)
        pltpu.async_copy(ew_hbm.at[pl.ds(row0, RSTD)],
                         ew_l.at[pl.ds(0, RSTD)], ssem)

    @pl.when(last)
    def _():
        pltpu.async_copy(ei_hbm.at[pl.ds(row0, RLAST), 0], src_l, ssem)
        pltpu.async_copy(ei_hbm.at[pl.ds(row0, RLAST), 1], dst_l, ssem)
        pltpu.async_copy(ew_hbm.at[pl.ds(row0, RLAST)], ew_l, ssem)
    pltpu.async_copy(y0_hbm, y_l.at[pl.ds(0, N)], ssem)

    def _zero(i, _):
        sbuf[pl.ds(i * VL, VL)] = jnp.zeros((VL,), jnp.float32)
        return 0

    lax.fori_loop(0, SLICE // VL, _zero, 0)

    @pl.when(jnp.logical_not(last))
    def _():
        for _i in range(2):
            pltpu.make_async_copy(ei_hbm.at[pl.ds(0, RSTD), 0],
                                  src_l.at[pl.ds(0, RSTD)], ssem).wait()
        pltpu.make_async_copy(ew_hbm.at[pl.ds(0, RSTD)],
                              ew_l.at[pl.ds(0, RSTD)], ssem).wait()

    @pl.when(last)
    def _():
        for _i in range(2):
            pltpu.make_async_copy(ei_hbm.at[pl.ds(0, RLAST), 0],
                                  src_l, ssem).wait()
        pltpu.make_async_copy(ew_hbm.at[pl.ds(0, RLAST)], ew_l, ssem).wait()
    pltpu.make_async_copy(y0_hbm, y_l.at[pl.ds(0, N)], ssem).wait()

    def _fill(j, _):
        for g in range(CHUNK // VL):
            src_l[j, pl.ds(g * VL, VL)] = jnp.zeros((VL,), jnp.int32)
            ew_l[j, pl.ds(g * VL, VL)] = jnp.zeros((VL,), jnp.float32)
            dst_l[j, pl.ds(g * VL, VL)] = jnp.full((VL,), N, jnp.int32)
        return 0

    lax.fori_loop(nreal, RPT, _fill, 0)

    pltpu.sync_copy(sbuf, acc0.at[pl.ds(sid * SLICE, SLICE)])
    pltpu.sync_copy(sbuf, acc1.at[pl.ds(sid * SLICE, SLICE)])
    plsc.subcore_barrier()

    for hop in range(K):
        acc = acc0 if hop == 0 else acc1

        def _row(j, _):
            for g in range(CHUNK // VL):
                s16 = src_l[j, pl.ds(g * VL, VL)]
                e16 = ew_l[j, pl.ds(g * VL, VL)]
                vals = plsc.load_gather(y_l, [s16])
                msg_f[pl.ds(j * CHUNK + g * VL, VL)] = vals * e16
                if hop == 0:
                    dst_f[pl.ds(j * CHUNK + g * VL, VL)] = \
                        dst_l[j, pl.ds(g * VL, VL)]
            return 0

        for c in range(NCH):
            lax.fori_loop(c * CROWS, (c + 1) * CROWS, _row, 0)
            pltpu.async_copy(
                msg_f.at[pl.ds(c * CSIZE, CSIZE)],
                acc.at[dst_f.at[pl.ds(c * CSIZE, CSIZE)]],
                sem, add=True)
        for c in range(NCH):
            pltpu.make_async_copy(
                msg_f.at[pl.ds(c * CSIZE, CSIZE)],
                acc.at[dst_f.at[pl.ds(c * CSIZE, CSIZE)]], sem).wait()
        plsc.subcore_barrier()
        if hop + 1 < K:
            @pl.when(jnp.logical_not(last))
            def _():
                pltpu.sync_copy(acc.at[pl.ds(sid * OSLICE, OSLICE)],
                                out_hbm.at[pl.ds(sid * OSLICE, OSLICE)])

            @pl.when(last)
            def _():
                pltpu.sync_copy(acc.at[pl.ds(15 * OSLICE, OLAST)],
                                out_hbm.at[pl.ds(15 * OSLICE, OLAST)])
            plsc.subcore_barrier()
            pltpu.sync_copy(out_hbm, y_l.at[pl.ds(0, N)])

    def _sig(i, _):
        z = sbuf[pl.ds(i * VL, VL)]
        sbuf[pl.ds(i * VL, VL)] = 1.0 / (1.0 + jnp.exp(-z))
        return 0

    @pl.when(jnp.logical_not(last))
    def _():
        pltpu.sync_copy(acc1.at[pl.ds(sid * OSLICE, OSLICE)],
                        sbuf.at[pl.ds(0, OSLICE)])
        lax.fori_loop(0, OSLICE // VL, _sig, 0)
        pltpu.sync_copy(sbuf.at[pl.ds(0, OSLICE)],
                        out_hbm.at[pl.ds(sid * OSLICE, OSLICE)])

    @pl.when(last)
    def _():
        pltpu.sync_copy(acc1.at[pl.ds(15 * OSLICE, OLAST)],
                        sbuf.at[pl.ds(0, OLAST)])
        lax.fori_loop(0, OLAST // VL, _sig, 0)
        pltpu.sync_copy(sbuf.at[pl.ds(0, OLAST)],
                        out_hbm.at[pl.ds(15 * OSLICE, OLAST)])


def kernel(x, edge_index, edge_weight, w):
    ei3 = edge_index.reshape(2, EROWS, CHUNK).transpose(1, 0, 2)
    ew2d = edge_weight.reshape(EROWS, CHUNK)
    y0 = _matvec(x, w.reshape(1, D))
    res = _sgc_sc(y0, ei3, ew2d)
    return res.reshape(N, 1)


# speedup vs baseline: 1.4819x; 1.2585x over previous
import functools

import jax
import jax.numpy as jnp
from jax import lax
from jax.experimental import pallas as pl
from jax.experimental.pallas import tpu as pltpu
from jax.experimental.pallas import tpu_sc as plsc

N = 10000
E = 320000
D = 128
K = 2

NS = 16
CHUNK = 128
EROWS = E // CHUNK
RPT = 157
RLAST = EROWS - 15 * RPT
EPT_PAD = RPT * CHUNK
NPAD = 10240
SLICE = NPAD // NS
VL = 16
SCHUNKS = [(0, 20), (20, 20), (40, 20), (60, 20),
           (80, 20), (100, 20), (120, 20), (140, 17)]


def _matvec_body(x_ref, w_ref, o_ref):
    o_ref[...] = jax.lax.dot_general(
        w_ref[...], x_ref[...],
        dimension_numbers=(((1,), (1,)), ((), ())),
        preferred_element_type=jnp.float32).reshape(N)


def _matvec(x, w2):
    return pl.pallas_call(
        _matvec_body,
        in_specs=[
            pl.BlockSpec((N, D), lambda: (0, 0)),
            pl.BlockSpec((1, D), lambda: (0, 0)),
        ],
        out_specs=pl.BlockSpec((N,), lambda: (0,)),
        out_shape=jax.ShapeDtypeStruct((N,), jnp.float32),
    )(x, w2)


_mesh = plsc.VectorSubcoreMesh(
    core_axis_name="c", subcore_axis_name="s", num_cores=1)


@functools.partial(
    pl.kernel,
    out_type=jax.ShapeDtypeStruct((NPAD,), jnp.float32),
    mesh=_mesh,
    compiler_params=pltpu.CompilerParams(
        use_tc_tiling_on_sc=False, needs_layout_passes=False),
    scratch_types=[
        pltpu.VMEM((NPAD,), jnp.float32),
        pltpu.VMEM((RPT, CHUNK), jnp.int32),
        pltpu.VMEM((RPT, CHUNK), jnp.int32),
        pltpu.VMEM((RPT, CHUNK), jnp.float32),
        pltpu.VMEM((EPT_PAD,), jnp.float32),
        pltpu.VMEM((EPT_PAD,), jnp.int32),
        pltpu.VMEM((SLICE,), jnp.float32),
        pltpu.VMEM_SHARED((NPAD,), jnp.float32),
        pltpu.VMEM_SHARED((NPAD,), jnp.float32),
        pltpu.SemaphoreType.DMA,
        pltpu.SemaphoreType.DMA,
    ],
)
def _sgc_sc(y0_hbm, ei_hbm, ew_hbm, out_hbm,
            y_l, src_l, dst_l, ew_l, msg_f, dst_f, sbuf,
            acc0, acc1, ssem, sem):
    sid = lax.axis_index("s")
    row0 = sid * RPT

    @pl.when(sid < NS - 1)
    def _():
        pltpu.async_copy(ei_hbm.at[pl.ds(row0, RPT), 0], src_l, ssem)
        pltpu.async_copy(ei_hbm.at[pl.ds(row0, RPT), 1], dst_l, ssem)
        pltpu.async_copy(ew_hbm.at[pl.ds(row0, RPT)], ew_l, ssem)

    @pl.when(sid == NS - 1)
    def _():
        pltpu.async_copy(ei_hbm.at[pl.ds(row0, RLAST), 0],
                         src_l.at[pl.ds(0, RLAST)], ssem)
        pltpu.async_copy(ei_hbm.at[pl.ds(row0, RLAST), 1],
                         dst_l.at[pl.ds(0, RLAST)], ssem)
        pltpu.async_copy(ew_hbm.at[pl.ds(row0, RLAST)],
                         ew_l.at[pl.ds(0, RLAST)], ssem)
    pltpu.async_copy(y0_hbm, y_l.at[pl.ds(0, N)], ssem)

    def _zero(i, _):
        sbuf[pl.ds(i * VL, VL)] = jnp.zeros((VL,), jnp.float32)
        return 0

    lax.fori_loop(0, SLICE // VL, _zero, 0)

    @pl.when(sid < NS - 1)
    def _():
        pltpu.make_async_copy(ei_hbm.at[pl.ds(0, RPT), 0], src_l, ssem).wait()
        pltpu.make_async_copy(ei_hbm.at[pl.ds(0, RPT), 1], dst_l, ssem).wait()
        pltpu.make_async_copy(ew_hbm.at[pl.ds(0, RPT)], ew_l, ssem).wait()

    @pl.when(sid == NS - 1)
    def _():
        pltpu.make_async_copy(ei_hbm.at[pl.ds(0, RLAST), 0],
                              src_l.at[pl.ds(0, RLAST)], ssem).wait()
        pltpu.make_async_copy(ei_hbm.at[pl.ds(0, RLAST), 1],
                              dst_l.at[pl.ds(0, RLAST)], ssem).wait()
        pltpu.make_async_copy(ew_hbm.at[pl.ds(0, RLAST)],
                              ew_l.at[pl.ds(0, RLAST)], ssem).wait()
    pltpu.make_async_copy(y0_hbm, y_l.at[pl.ds(0, N)], ssem).wait()

    @pl.when(sid == NS - 1)
    def _():
        def _fill(j, _):
            for g in range(CHUNK // VL):
                src_l[j, pl.ds(g * VL, VL)] = jnp.zeros((VL,), jnp.int32)
                ew_l[j, pl.ds(g * VL, VL)] = jnp.zeros((VL,), jnp.float32)
                dst_l[j, pl.ds(g * VL, VL)] = jnp.full((VL,), N, jnp.int32)
            return 0
        lax.fori_loop(RLAST, RPT, _fill, 0)

    pltpu.sync_copy(sbuf, acc0.at[pl.ds(sid * SLICE, SLICE)])
    pltpu.sync_copy(sbuf, acc1.at[pl.ds(sid * SLICE, SLICE)])
    plsc.subcore_barrier()

    for hop in range(K):
        acc = acc0 if hop == 0 else acc1

        def _row(j, _):
            for g in range(CHUNK // VL):
                s16 = src_l[j, pl.ds(g * VL, VL)]
                e16 = ew_l[j, pl.ds(g * VL, VL)]
                vals = plsc.load_gather(y_l, [s16])
                msg_f[pl.ds(j * CHUNK + g * VL, VL)] = vals * e16
                if hop == 0:
                    dst_f[pl.ds(j * CHUNK + g * VL, VL)] = \
                        dst_l[j, pl.ds(g * VL, VL)]
            return 0

        for (r0, rn) in SCHUNKS:
            lax.fori_loop(r0, r0 + rn, _row, 0)
            pltpu.async_copy(
                msg_f.at[pl.ds(r0 * CHUNK, rn * CHUNK)],
                acc.at[dst_f.at[pl.ds(r0 * CHUNK, rn * CHUNK)]],
                sem, add=True)
        for (r0, rn) in SCHUNKS:
            pltpu.make_async_copy(
                msg_f.at[pl.ds(r0 * CHUNK, rn * CHUNK)],
                acc.at[dst_f.at[pl.ds(r0 * CHUNK, rn * CHUNK)]],
                sem).wait()
        plsc.subcore_barrier()
        if hop + 1 < K:
            pltpu.sync_copy(acc.at[pl.ds(sid * SLICE, SLICE)],
                            out_hbm.at[pl.ds(sid * SLICE, SLICE)])
            plsc.subcore_barrier()
            pltpu.sync_copy(out_hbm, y_l)

    pltpu.sync_copy(acc1.at[pl.ds(sid * SLICE, SLICE)], sbuf)

    def _sig(i, _):
        z = sbuf[pl.ds(i * VL, VL)]
        sbuf[pl.ds(i * VL, VL)] = 1.0 / (1.0 + jnp.exp(-z))
        return 0

    lax.fori_loop(0, SLICE // VL, _sig, 0)
    pltpu.sync_copy(sbuf, out_hbm.at[pl.ds(sid * SLICE, SLICE)])


def kernel(x, edge_index, edge_weight, w):
    ei3 = edge_index.reshape(2, EROWS, CHUNK).transpose(1, 0, 2)
    ew2d = edge_weight.reshape(EROWS, CHUNK)
    y0 = _matvec(x, w.reshape(1, D))
    res = _sgc_sc(y0, ei3, ew2d)
    return res[:N].reshape(N, 1)
